# Initial kernel scaffold; baseline (speedup 1.0000x reference)
import jax
import jax.numpy as jnp
from jax.experimental import pallas as pl


def kernel(feat_rows, feat_cols, feat_vals, adj_rows, adj_cols, adj_vals, adj_var, features_nonzero, W0_mean, W1_mean, W0_var, W1_var):
    raise NotImplementedError("write your pallas kernel here")


# speedup vs baseline: 4.1405x; 4.1405x over previous
import functools

import jax
import jax.numpy as jnp
from jax import lax
from jax.experimental import pallas as pl
from jax.experimental.pallas import tpu as pltpu
from jax.experimental.pallas import tpu_sc as plsc

N = 10000
NF = 128
H = 64
OUT = 32

NPAD = 10240
STRIPE = NPAD // 16
BK = 128

_i32 = jnp.int32


def _pad_edges(rows, cols, vals, per_worker_mult, n_workers, n_cols):
    e = rows.shape[0]
    per_worker = -(-e // (n_workers * per_worker_mult)) * per_worker_mult
    total = per_worker * n_workers
    pad = total - e
    if pad:
        ar = jnp.arange(pad, dtype=_i32)
        rows = jnp.concatenate([rows, N + (ar % 32)])
        cols = jnp.concatenate([cols, ar % n_cols])
        vals = jnp.concatenate([vals, jnp.zeros((pad,), jnp.float32)])
    return rows, cols, vals, per_worker


_DNUMS = lax.GatherDimensionNumbers(
    offset_dims=(), collapsed_slice_dims=(0,), start_index_map=(0,))


def _bcast(vec16, lane):
    idx = jnp.full((16, 1), lane, dtype=_i32)
    return lax.gather(vec16, idx, _DNUMS, (1,),
                      mode=lax.GatherScatterMode.PROMISE_IN_BOUNDS)


def _scale_window(gath_v, vals_v, width):
    for eb in range(BK // 16):
        vals16 = vals_v[pl.ds(eb * 16, 16)]
        for l in range(16):
            e = eb * 16 + l
            val16 = _bcast(vals16, l)
            for j in range(width // 16):
                g = gath_v[e, pl.ds(j * 16, 16)]
                gath_v[e, pl.ds(j * 16, 16)] = g * val16


def _feat_window(gath_v, idx_v, vals_v, table_v, width):
    base = lax.iota(_i32, 16)
    for eb in range(BK // 16):
        vals16 = vals_v[pl.ds(eb * 16, 16)]
        cols16 = idx_v[pl.ds(eb * 16, 16)]
        for l in range(16):
            e = eb * 16 + l
            val16 = _bcast(vals16, l)
            rowoff = _bcast(cols16, l) * width + base
            for j in range(width // 16):
                g = plsc.load_gather(table_v, [rowoff + (j * 16)])
                gath_v[e, pl.ds(j * 16, 16)] = g * val16


def _zero_acc(gath_v, acc_shared, sid, width):
    z = jnp.zeros((16,), jnp.float32)
    for r in range(BK):
        for j in range(width // 16):
            gath_v[r, pl.ds(j * 16, 16)] = z
    for k in range(STRIPE // BK):
        pltpu.sync_copy(gath_v, acc_shared.at[pl.ds(sid * STRIPE + k * BK, BK)])


def _make_adj_spmm(per_worker, width):
    n_win = per_worker // BK
    mesh = plsc.VectorSubcoreMesh(core_axis_name="c", subcore_axis_name="s")

    @functools.partial(
        pl.kernel,
        mesh=mesh,
        out_type=jax.ShapeDtypeStruct((2, NPAD, width), jnp.float32),
        scratch_types=[
            pltpu.VMEM((BK,), _i32),
            pltpu.VMEM((BK,), _i32),
            pltpu.VMEM((BK,), jnp.float32),
            pltpu.VMEM((BK, width), jnp.float32),
            pltpu.VMEM_SHARED((NPAD, width), jnp.float32),
            pltpu.SemaphoreType.DMA,
        ],
        compiler_params=pltpu.CompilerParams(needs_layout_passes=False, use_tc_tiling_on_sc=False),
    )
    def adj_spmm(cols_hbm, vals_hbm, rows_hbm, x_hbm, out_hbm,
                 idx_v, rows_v, vals_v, gath_v, acc_shared, sem):
        cid = lax.axis_index("c")
        sid = lax.axis_index("s")
        _zero_acc(gath_v, acc_shared, sid, width)
        plsc.subcore_barrier()

        chunk = (cid * 16 + sid) * per_worker

        def body(w, _):
            off = chunk + w * BK
            pltpu.sync_copy(cols_hbm.at[pl.ds(off, BK)], idx_v)
            pltpu.sync_copy(vals_hbm.at[pl.ds(off, BK)], vals_v)
            pltpu.sync_copy(rows_hbm.at[pl.ds(off, BK)], rows_v)
            pltpu.async_copy(x_hbm.at[idx_v], gath_v, sem).wait()
            _scale_window(gath_v, vals_v, width)
            pltpu.sync_copy(gath_v, acc_shared.at[rows_v], add=True)
            return _

        lax.fori_loop(0, n_win, body, None)
        plsc.subcore_barrier()
        pltpu.sync_copy(
            acc_shared.at[pl.ds(sid * STRIPE, STRIPE)],
            out_hbm.at[cid, pl.ds(sid * STRIPE, STRIPE)],
        )

    return adj_spmm


def _make_feat_spmm(per_worker, width):
    n_win = per_worker // BK
    mesh = plsc.VectorSubcoreMesh(core_axis_name="c", subcore_axis_name="s")

    @functools.partial(
        pl.kernel,
        mesh=mesh,
        out_type=jax.ShapeDtypeStruct((2, NPAD, width), jnp.float32),
        scratch_types=[
            pltpu.VMEM((BK,), _i32),
            pltpu.VMEM((BK,), _i32),
            pltpu.VMEM((BK,), jnp.float32),
            pltpu.VMEM((BK, width), jnp.float32),
            pltpu.VMEM((NF * width,), jnp.float32),
            pltpu.VMEM_SHARED((NPAD, width), jnp.float32),
        ],
        compiler_params=pltpu.CompilerParams(needs_layout_passes=False, use_tc_tiling_on_sc=False),
    )
    def feat_spmm(cols_hbm, vals_hbm, rows_hbm, w0_hbm, out_hbm,
                  idx_v, rows_v, vals_v, gath_v, table_v, acc_shared):
        cid = lax.axis_index("c")
        sid = lax.axis_index("s")
        pltpu.sync_copy(w0_hbm.at[cid], table_v)
        _zero_acc(gath_v, acc_shared, sid, width)
        plsc.subcore_barrier()

        chunk = sid * per_worker

        def body(w, _):
            off = chunk + w * BK
            pltpu.sync_copy(cols_hbm.at[pl.ds(off, BK)], idx_v)
            pltpu.sync_copy(vals_hbm.at[pl.ds(off, BK)], vals_v)
            pltpu.sync_copy(rows_hbm.at[pl.ds(off, BK)], rows_v)
            _feat_window(gath_v, idx_v, vals_v, table_v, width)
            pltpu.sync_copy(gath_v, acc_shared.at[rows_v], add=True)
            return _

        lax.fori_loop(0, n_win, body, None)
        plsc.subcore_barrier()
        pltpu.sync_copy(
            acc_shared.at[pl.ds(sid * STRIPE, STRIPE)],
            out_hbm.at[cid, pl.ds(sid * STRIPE, STRIPE)],
        )

    return feat_spmm


_RB = 1000


def _tc_mid_body(hp0, hp1, xv, av, w1m, w1v, hw_ref, lv_ref):
    h = jnp.maximum(hp0[...] + hp1[...], 0.0)
    hw_ref[...] = jnp.dot(h, w1m[...], preferred_element_type=jnp.float32)
    xvr = jnp.maximum(xv[...] * av[...], 0.0)
    lv_ref[...] = (
        jnp.dot(xvr, w1v[...], preferred_element_type=jnp.float32) * av[...]
    )


def _tc_mid(hp0, hp1, xv, av, w1m, w1v):
    grid = (N // _RB,)
    row_spec = lambda w: pl.BlockSpec((_RB, w), lambda i: (i, 0))
    full_spec = lambda a, b: pl.BlockSpec((a, b), lambda i: (0, 0))
    return pl.pallas_call(
        _tc_mid_body,
        grid=grid,
        in_specs=[row_spec(H), row_spec(H), row_spec(H),
                  pl.BlockSpec((_RB, 1), lambda i: (i, 0)),
                  full_spec(H, OUT), full_spec(H, OUT)],
        out_specs=[row_spec(OUT), row_spec(OUT)],
        out_shape=[jax.ShapeDtypeStruct((N, OUT), jnp.float32),
                   jax.ShapeDtypeStruct((N, OUT), jnp.float32)],
    )(hp0, hp1, xv, av, w1m, w1v)


def _tc_add_body(a, b, o):
    o[...] = a[...] + b[...]


def _tc_add(a, b):
    spec = pl.BlockSpec((_RB, OUT), lambda i: (i, 0))
    return pl.pallas_call(
        _tc_add_body,
        grid=(N // _RB,),
        in_specs=[spec, spec],
        out_specs=spec,
        out_shape=jax.ShapeDtypeStruct((N, OUT), jnp.float32),
    )(a, b)


def kernel(feat_rows, feat_cols, feat_vals, adj_rows, adj_cols, adj_vals,
           adj_var, features_nonzero, W0_mean, W1_mean, W0_var, W1_var):
    del features_nonzero
    fr = feat_rows.astype(_i32)
    fc = feat_cols.astype(_i32)
    ar = adj_rows.astype(_i32)
    ac = adj_cols.astype(_i32)

    fr, fc, fv, pw_f = _pad_edges(fr, fc, feat_vals, BK, 16, NF)
    w0 = jnp.stack([W0_mean, W0_var]).reshape(2, NF * H)
    xcat = _make_feat_spmm(pw_f, H)(fc, fv, fr, w0)
    x, xv = xcat[0, :N], xcat[1, :N]

    ar, ac, av, pw_a = _pad_edges(ar, ac, adj_vals, BK, 32, N)
    hp = _make_adj_spmm(pw_a, H)(ac, av, ar, x)

    av2 = adj_var[:, None]
    hw, log_var = _tc_mid(hp[0, :N], hp[1, :N], xv, av2, W1_mean, W1_var)

    mp = _make_adj_spmm(pw_a, OUT)(ac, av, ar, hw)
    m = _tc_add(mp[0, :N], mp[1, :N])
    return m, log_var


# speedup vs baseline: 6.9250x; 1.6725x over previous
import functools

import jax
import jax.numpy as jnp
from jax import lax
from jax.experimental import pallas as pl
from jax.experimental.pallas import tpu as pltpu
from jax.experimental.pallas import tpu_sc as plsc

N = 10000
NF = 128
H = 64
OUT = 32

NPAD = 10240
STRIPE = NPAD // 16
BK = 128

_i32 = jnp.int32


def _pad_edges(rows, cols, vals, per_worker_mult, n_workers, n_cols):
    e = rows.shape[0]
    per_worker = -(-e // (n_workers * per_worker_mult)) * per_worker_mult
    total = per_worker * n_workers
    pad = total - e
    if pad:
        ar = jnp.arange(pad, dtype=_i32)
        rows = jnp.concatenate([rows, N + (ar % 32)])
        cols = jnp.concatenate([cols, ar % n_cols])
        vals = jnp.concatenate([vals, jnp.zeros((pad,), jnp.float32)])
    return rows, cols, vals, per_worker


_DNUMS = lax.GatherDimensionNumbers(
    offset_dims=(), collapsed_slice_dims=(0,), start_index_map=(0,))


def _bcast(vec16, lane):
    idx = jnp.full((16, 1), lane, dtype=_i32)
    return lax.gather(vec16, idx, _DNUMS, (1,),
                      mode=lax.GatherScatterMode.PROMISE_IN_BOUNDS)


def _copy_window(src_all, woff, dst_v):
    for eb in range(BK // 16):
        dst_v[pl.ds(eb * 16, 16)] = src_all[pl.ds(woff + eb * 16, 16)]


def _scale_window(gath_v, vals_all, woff, width):
    for eb in range(BK // 16):
        vals16 = vals_all[pl.ds(woff + eb * 16, 16)]
        for l in range(16):
            e = eb * 16 + l
            val16 = _bcast(vals16, l)
            for j in range(width // 16):
                g = gath_v[e, pl.ds(j * 16, 16)]
                gath_v[e, pl.ds(j * 16, 16)] = g * val16


def _feat_window(gath_v, cols_all, vals_all, woff, table_v, width):
    base = lax.iota(_i32, 16)
    for eb in range(BK // 16):
        vals16 = vals_all[pl.ds(woff + eb * 16, 16)]
        cols16 = cols_all[pl.ds(woff + eb * 16, 16)]
        for l in range(16):
            e = eb * 16 + l
            val16 = _bcast(vals16, l)
            rowoff = _bcast(cols16, l) * width + base
            for j in range(width // 16):
                g = plsc.load_gather(table_v, [rowoff + (j * 16)])
                gath_v[e, pl.ds(j * 16, 16)] = g * val16


def _zero_acc(buf_v, acc_shared, sid, width):
    z = jnp.zeros((16,), jnp.float32)
    for r in range(BK):
        for j in range(width // 16):
            buf_v[r, pl.ds(j * 16, 16)] = z
    for k in range(STRIPE // BK):
        pltpu.sync_copy(buf_v, acc_shared.at[pl.ds(sid * STRIPE + k * BK, BK)])


def _make_adj_spmm(per_worker, width):
    n_win = per_worker // BK
    assert n_win % 2 == 0
    mesh = plsc.VectorSubcoreMesh(core_axis_name="c", subcore_axis_name="s")

    @functools.partial(
        pl.kernel,
        mesh=mesh,
        out_type=jax.ShapeDtypeStruct((2, NPAD, width), jnp.float32),
        scratch_types=[
            pltpu.VMEM((per_worker,), _i32),
            pltpu.VMEM((per_worker,), _i32),
            pltpu.VMEM((per_worker,), jnp.float32),
            pltpu.VMEM((BK,), _i32),
            pltpu.VMEM((BK,), _i32),
            pltpu.VMEM((BK,), _i32),
            pltpu.VMEM((BK, width), jnp.float32),
            pltpu.VMEM((BK, width), jnp.float32),
            pltpu.VMEM_SHARED((NPAD, width), jnp.float32),
            pltpu.SemaphoreType.DMA,
            pltpu.SemaphoreType.DMA,
        ],
        compiler_params=pltpu.CompilerParams(
            needs_layout_passes=False, use_tc_tiling_on_sc=False),
    )
    def adj_spmm(cols_hbm, vals_hbm, rows_hbm, x_hbm, out_hbm,
                 cols_all, rows_all, vals_all, idx0_v, idx1_v, rows_v,
                 gath0_v, gath1_v, acc_shared, sem0, sem1):
        cid = lax.axis_index("c")
        sid = lax.axis_index("s")
        chunk = (cid * 16 + sid) * per_worker
        pltpu.sync_copy(cols_hbm.at[pl.ds(chunk, per_worker)], cols_all)
        pltpu.sync_copy(vals_hbm.at[pl.ds(chunk, per_worker)], vals_all)
        pltpu.sync_copy(rows_hbm.at[pl.ds(chunk, per_worker)], rows_all)
        _zero_acc(gath0_v, acc_shared, sid, width)
        plsc.subcore_barrier()

        idx_bufs = (idx0_v, idx1_v)
        gath_bufs = (gath0_v, gath1_v)
        sems = (sem0, sem1)

        for b in range(2):
            _copy_window(cols_all, b * BK, idx_bufs[b])
            pltpu.async_copy(x_hbm.at[idx_bufs[b]], gath_bufs[b], sems[b])

        def body(k, _):
            for b in range(2):
                w = k * 2 + b
                woff = w * BK
                pltpu.make_async_copy(
                    x_hbm.at[idx_bufs[b]], gath_bufs[b], sems[b]).wait()
                _scale_window(gath_bufs[b], vals_all, woff, width)
                _copy_window(rows_all, woff, rows_v)
                pltpu.sync_copy(gath_bufs[b], acc_shared.at[rows_v], add=True)

                @pl.when(w + 2 < n_win)
                def _():
                    _copy_window(cols_all, woff + 2 * BK, idx_bufs[b])
                    pltpu.async_copy(
                        x_hbm.at[idx_bufs[b]], gath_bufs[b], sems[b])
            return _

        lax.fori_loop(0, n_win // 2, body, None)
        plsc.subcore_barrier()
        pltpu.sync_copy(
            acc_shared.at[pl.ds(sid * STRIPE, STRIPE)],
            out_hbm.at[cid, pl.ds(sid * STRIPE, STRIPE)],
        )

    return adj_spmm


def _make_feat_spmm(per_worker, width):
    n_win = per_worker // BK
    mesh = plsc.VectorSubcoreMesh(core_axis_name="c", subcore_axis_name="s")

    @functools.partial(
        pl.kernel,
        mesh=mesh,
        out_type=jax.ShapeDtypeStruct((2, NPAD, width), jnp.float32),
        scratch_types=[
            pltpu.VMEM((per_worker,), _i32),
            pltpu.VMEM((per_worker,), _i32),
            pltpu.VMEM((per_worker,), jnp.float32),
            pltpu.VMEM((BK,), _i32),
            pltpu.VMEM((BK, width), jnp.float32),
            pltpu.VMEM((NF * width,), jnp.float32),
            pltpu.VMEM_SHARED((NPAD, width), jnp.float32),
        ],
        compiler_params=pltpu.CompilerParams(
            needs_layout_passes=False, use_tc_tiling_on_sc=False),
    )
    def feat_spmm(cols_hbm, vals_hbm, rows_hbm, w0_hbm, out_hbm,
                  cols_all, rows_all, vals_all, rows_v, gath_v, table_v,
                  acc_shared):
        cid = lax.axis_index("c")
        sid = lax.axis_index("s")
        chunk = sid * per_worker
        pltpu.sync_copy(w0_hbm.at[cid], table_v)
        pltpu.sync_copy(cols_hbm.at[pl.ds(chunk, per_worker)], cols_all)
        pltpu.sync_copy(vals_hbm.at[pl.ds(chunk, per_worker)], vals_all)
        pltpu.sync_copy(rows_hbm.at[pl.ds(chunk, per_worker)], rows_all)
        _zero_acc(gath_v, acc_shared, sid, width)
        plsc.subcore_barrier()

        def body(w, _):
            woff = w * BK
            _feat_window(gath_v, cols_all, vals_all, woff, table_v, width)
            _copy_window(rows_all, woff, rows_v)
            pltpu.sync_copy(gath_v, acc_shared.at[rows_v], add=True)
            return _

        lax.fori_loop(0, n_win, body, None)
        plsc.subcore_barrier()
        pltpu.sync_copy(
            acc_shared.at[pl.ds(sid * STRIPE, STRIPE)],
            out_hbm.at[cid, pl.ds(sid * STRIPE, STRIPE)],
        )

    return feat_spmm


_RB = 1000


def _tc_mid_body(hp0, hp1, xv, av, w1m, w1v, hw_ref, lv_ref):
    h = jnp.maximum(hp0[...] + hp1[...], 0.0)
    hw_ref[...] = jnp.dot(h, w1m[...], preferred_element_type=jnp.float32)
    xvr = jnp.maximum(xv[...] * av[...], 0.0)
    lv_ref[...] = (
        jnp.dot(xvr, w1v[...], preferred_element_type=jnp.float32) * av[...]
    )


def _tc_mid(hp0, hp1, xv, av, w1m, w1v):
    grid = (N // _RB,)
    row_spec = lambda w: pl.BlockSpec((_RB, w), lambda i: (i, 0))
    full_spec = lambda a, b: pl.BlockSpec((a, b), lambda i: (0, 0))
    return pl.pallas_call(
        _tc_mid_body,
        grid=grid,
        in_specs=[row_spec(H), row_spec(H), row_spec(H),
                  pl.BlockSpec((_RB, 1), lambda i: (i, 0)),
                  full_spec(H, OUT), full_spec(H, OUT)],
        out_specs=[row_spec(OUT), row_spec(OUT)],
        out_shape=[jax.ShapeDtypeStruct((N, OUT), jnp.float32),
                   jax.ShapeDtypeStruct((N, OUT), jnp.float32)],
    )(hp0, hp1, xv, av, w1m, w1v)


def _tc_add_body(a, b, o):
    o[...] = a[...] + b[...]


def _tc_add(a, b):
    spec = pl.BlockSpec((_RB, OUT), lambda i: (i, 0))
    return pl.pallas_call(
        _tc_add_body,
        grid=(N // _RB,),
        in_specs=[spec, spec],
        out_specs=spec,
        out_shape=jax.ShapeDtypeStruct((N, OUT), jnp.float32),
    )(a, b)


def kernel(feat_rows, feat_cols, feat_vals, adj_rows, adj_cols, adj_vals,
           adj_var, features_nonzero, W0_mean, W1_mean, W0_var, W1_var):
    del features_nonzero
    fr = feat_rows.astype(_i32)
    fc = feat_cols.astype(_i32)
    ar = adj_rows.astype(_i32)
    ac = adj_cols.astype(_i32)

    fr, fc, fv, pw_f = _pad_edges(fr, fc, feat_vals, BK, 16, NF)
    w0 = jnp.stack([W0_mean, W0_var]).reshape(2, NF * H)
    xcat = _make_feat_spmm(pw_f, H)(fc, fv, fr, w0)
    x, xv = xcat[0, :N], xcat[1, :N]

    ar, ac, av, pw_a = _pad_edges(ar, ac, adj_vals, 2 * BK, 32, N)
    hp = _make_adj_spmm(pw_a, H)(ac, av, ar, x)

    av2 = adj_var[:, None]
    hw, log_var = _tc_mid(hp[0, :N], hp[1, :N], xv, av2, W1_mean, W1_var)

    mp = _make_adj_spmm(pw_a, OUT)(ac, av, ar, hw)
    m = _tc_add(mp[0, :N], mp[1, :N])
    return m, log_var


# speedup vs baseline: 13.5946x; 1.9631x over previous
import functools

import jax
import jax.numpy as jnp
from jax import lax
from jax.experimental import pallas as pl
from jax.experimental.pallas import tpu as pltpu
from jax.experimental.pallas import tpu_sc as plsc

N = 10000
NF = 128
H = 64
OUT = 32

NPAD = 10240
STRIPE = NPAD // 16
BK = 128

_i32 = jnp.int32


def _pad_edges(rows, cols, vals, per_worker_mult, n_workers, n_cols):
    e = rows.shape[0]
    per_worker = -(-e // (n_workers * per_worker_mult)) * per_worker_mult
    total = per_worker * n_workers
    pad = total - e
    if pad:
        ar = jnp.arange(pad, dtype=_i32)
        rows = jnp.concatenate([rows, N + (ar % 32)])
        cols = jnp.concatenate([cols, ar % n_cols])
        vals = jnp.concatenate([vals, jnp.zeros((pad,), jnp.float32)])
    return rows, cols, vals, per_worker


_DNUMS = lax.GatherDimensionNumbers(
    offset_dims=(), collapsed_slice_dims=(0,), start_index_map=(0,))


def _bcast(vec16, lane):
    idx = jnp.full((16, 1), lane, dtype=_i32)
    return lax.gather(vec16, idx, _DNUMS, (1,),
                      mode=lax.GatherScatterMode.PROMISE_IN_BOUNDS)


def _copy_window(src_all, woff, dst_v):
    for eb in range(BK // 16):
        dst_v[pl.ds(eb * 16, 16)] = src_all[pl.ds(woff + eb * 16, 16)]


def _scale_window(gath_v, scat_v, vals_all, woff, width):
    for eb in range(BK // 16):
        vals16 = vals_all[pl.ds(woff + eb * 16, 16)]
        for l in range(16):
            e = eb * 16 + l
            val16 = _bcast(vals16, l)
            for j in range(width // 16):
                g = gath_v[e, pl.ds(j * 16, 16)]
                scat_v[e, pl.ds(j * 16, 16)] = g * val16


def _zero_acc(buf_v, acc_shared, sid, width):
    z = jnp.zeros((16,), jnp.float32)
    for r in range(BK):
        for j in range(width // 16):
            buf_v[r, pl.ds(j * 16, 16)] = z
    for k in range(STRIPE // BK):
        pltpu.sync_copy(buf_v, acc_shared.at[pl.ds(sid * STRIPE + k * BK, BK)])


def _make_spmm(per_worker, width, feat_mode):
    n_win = per_worker // BK
    assert n_win % 2 == 0 and n_win >= 4
    mesh = plsc.VectorSubcoreMesh(core_axis_name="c", subcore_axis_name="s")

    scratch = [
        pltpu.VMEM((per_worker,), _i32),
        pltpu.VMEM((per_worker,), _i32),
        pltpu.VMEM((per_worker,), jnp.float32),
        pltpu.VMEM((BK,), _i32),
        pltpu.VMEM((BK,), _i32),
        pltpu.VMEM((BK,), _i32),
        pltpu.VMEM((BK,), _i32),
        pltpu.VMEM((BK, width), jnp.float32),
        pltpu.VMEM((BK, width), jnp.float32),
        pltpu.VMEM((BK, width), jnp.float32),
        pltpu.VMEM((BK, width), jnp.float32),
        pltpu.VMEM_SHARED((NPAD, width), jnp.float32),
        pltpu.SemaphoreType.DMA,
        pltpu.SemaphoreType.DMA,
        pltpu.SemaphoreType.DMA,
        pltpu.SemaphoreType.DMA,
    ]
    if feat_mode:
        scratch.append(pltpu.VMEM_SHARED((NF, width), jnp.float32))

    @functools.partial(
        pl.kernel,
        mesh=mesh,
        out_type=jax.ShapeDtypeStruct((2, NPAD, width), jnp.float32),
        scratch_types=scratch,
        compiler_params=pltpu.CompilerParams(
            needs_layout_passes=False, use_tc_tiling_on_sc=False),
    )
    def spmm(cols_hbm, vals_hbm, rows_hbm, x_hbm, out_hbm,
             cols_all, rows_all, vals_all, idx0_v, idx1_v, rows0_v, rows1_v,
             gath0_v, gath1_v, scat0_v, scat1_v, acc_shared,
             semg0, semg1, sems0, sems1, *rest):
        cid = lax.axis_index("c")
        sid = lax.axis_index("s")
        if feat_mode:
            table_sh = rest[0]
            chunk = sid * per_worker

            @pl.when(sid == 0)
            def _():
                pltpu.sync_copy(x_hbm.at[cid], table_sh)

            gsrc = table_sh
        else:
            chunk = (cid * 16 + sid) * per_worker
            gsrc = x_hbm
        pltpu.sync_copy(cols_hbm.at[pl.ds(chunk, per_worker)], cols_all)
        pltpu.sync_copy(vals_hbm.at[pl.ds(chunk, per_worker)], vals_all)
        pltpu.sync_copy(rows_hbm.at[pl.ds(chunk, per_worker)], rows_all)
        _zero_acc(gath0_v, acc_shared, sid, width)
        plsc.subcore_barrier()

        idx_bufs = (idx0_v, idx1_v)
        rows_bufs = (rows0_v, rows1_v)
        gath_bufs = (gath0_v, gath1_v)
        scat_bufs = (scat0_v, scat1_v)
        semg = (semg0, semg1)
        sems = (sems0, sems1)

        for b in range(2):
            _copy_window(cols_all, b * BK, idx_bufs[b])
            pltpu.async_copy(gsrc.at[idx_bufs[b]], gath_bufs[b], semg[b])

        def body(k, _):
            for b in range(2):
                w = k * 2 + b
                woff = w * BK
                pltpu.make_async_copy(
                    gsrc.at[idx_bufs[b]], gath_bufs[b], semg[b]).wait()

                @pl.when(k >= 1)
                def _():
                    pltpu.make_async_copy(
                        scat_bufs[b], acc_shared.at[rows_bufs[b]],
                        sems[b]).wait()

                _scale_window(gath_bufs[b], scat_bufs[b], vals_all, woff,
                              width)

                @pl.when(w + 2 < n_win)
                def _():
                    _copy_window(cols_all, woff + 2 * BK, idx_bufs[b])
                    pltpu.async_copy(
                        gsrc.at[idx_bufs[b]], gath_bufs[b], semg[b])

                _copy_window(rows_all, woff, rows_bufs[b])
                pltpu.async_copy(
                    scat_bufs[b], acc_shared.at[rows_bufs[b]], sems[b],
                    add=True)
            return _

        lax.fori_loop(0, n_win // 2, body, None)
        for b in range(2):
            pltpu.make_async_copy(
                scat_bufs[b], acc_shared.at[rows_bufs[b]], sems[b]).wait()
        plsc.subcore_barrier()
        pltpu.sync_copy(
            acc_shared.at[pl.ds(sid * STRIPE, STRIPE)],
            out_hbm.at[cid, pl.ds(sid * STRIPE, STRIPE)],
        )

    return spmm


_RB = 1000


def _tc_mid_body(hp0, hp1, xv, av, w1m, w1v, hw_ref, lv_ref):
    h = jnp.maximum(hp0[...] + hp1[...], 0.0)
    hw_ref[...] = jnp.dot(h, w1m[...], preferred_element_type=jnp.float32)
    xvr = jnp.maximum(xv[...] * av[...], 0.0)
    lv_ref[...] = (
        jnp.dot(xvr, w1v[...], preferred_element_type=jnp.float32) * av[...]
    )


def _tc_mid(hp0, hp1, xv, av, w1m, w1v):
    grid = (N // _RB,)
    row_spec = lambda w: pl.BlockSpec((_RB, w), lambda i: (i, 0))
    full_spec = lambda a, b: pl.BlockSpec((a, b), lambda i: (0, 0))
    return pl.pallas_call(
        _tc_mid_body,
        grid=grid,
        in_specs=[row_spec(H), row_spec(H), row_spec(H),
                  pl.BlockSpec((_RB, 1), lambda i: (i, 0)),
                  full_spec(H, OUT), full_spec(H, OUT)],
        out_specs=[row_spec(OUT), row_spec(OUT)],
        out_shape=[jax.ShapeDtypeStruct((N, OUT), jnp.float32),
                   jax.ShapeDtypeStruct((N, OUT), jnp.float32)],
    )(hp0, hp1, xv, av, w1m, w1v)


def _tc_add_body(a, b, o):
    o[...] = a[...] + b[...]


def _tc_add(a, b):
    spec = pl.BlockSpec((_RB, OUT), lambda i: (i, 0))
    return pl.pallas_call(
        _tc_add_body,
        grid=(N // _RB,),
        in_specs=[spec, spec],
        out_specs=spec,
        out_shape=jax.ShapeDtypeStruct((N, OUT), jnp.float32),
    )(a, b)


def kernel(feat_rows, feat_cols, feat_vals, adj_rows, adj_cols, adj_vals,
           adj_var, features_nonzero, W0_mean, W1_mean, W0_var, W1_var):
    del features_nonzero
    fr = feat_rows.astype(_i32)
    fc = feat_cols.astype(_i32)
    ar = adj_rows.astype(_i32)
    ac = adj_cols.astype(_i32)

    fr, fc, fv, pw_f = _pad_edges(fr, fc, feat_vals, 2 * BK, 16, NF)
    w0 = jnp.stack([W0_mean, W0_var])
    xcat = _make_spmm(pw_f, H, True)(fc, fv, fr, w0)
    x, xv = xcat[0, :N], xcat[1, :N]

    ar, ac, av, pw_a = _pad_edges(ar, ac, adj_vals, 2 * BK, 32, N)
    hp = _make_spmm(pw_a, H, False)(ac, av, ar, x)

    av2 = adj_var[:, None]
    hw, log_var = _tc_mid(hp[0, :N], hp[1, :N], xv, av2, W1_mean, W1_var)

    mp = _make_spmm(pw_a, OUT, False)(ac, av, ar, hw)
    m = _tc_add(mp[0, :N], mp[1, :N])
    return m, log_var
